# Initial kernel scaffold; baseline (speedup 1.0000x reference)
import jax
import jax.numpy as jnp
from jax.experimental import pallas as pl


def kernel(x, pe):
    raise NotImplementedError("write your pallas kernel here")


# speedup vs baseline: 1.5000x; 1.5000x over previous
import jax
import jax.numpy as jnp
from jax.experimental import pallas as pl


L_BLK = 512


def _body(x_ref, pe_ref, o_ref):
    o_ref[0] = x_ref[0] + pe_ref[...]


def kernel(x, pe):
    B, L, D = x.shape
    n_l = L // L_BLK
    return pl.pallas_call(
        _body,
        grid=(n_l, B),
        in_specs=[
            pl.BlockSpec((1, L_BLK, D), lambda i, b: (b, i, 0)),
            pl.BlockSpec((L_BLK, D), lambda i, b: (i, 0)),
        ],
        out_specs=pl.BlockSpec((1, L_BLK, D), lambda i, b: (b, i, 0)),
        out_shape=jax.ShapeDtypeStruct((B, L, D), x.dtype),
    )(x, pe[:L])


# speedup vs baseline: 1.6686x; 1.1124x over previous
import jax
import jax.numpy as jnp
from jax.experimental import pallas as pl


L_BLK = 1024


def _body(x_ref, pe_ref, o_ref):
    o_ref[0] = x_ref[0] + pe_ref[...]


def kernel(x, pe):
    B, L, D = x.shape
    n_l = L // L_BLK
    return pl.pallas_call(
        _body,
        grid=(n_l, B),
        in_specs=[
            pl.BlockSpec((1, L_BLK, D), lambda i, b: (b, i, 0)),
            pl.BlockSpec((L_BLK, D), lambda i, b: (i, 0)),
        ],
        out_specs=pl.BlockSpec((1, L_BLK, D), lambda i, b: (b, i, 0)),
        out_shape=jax.ShapeDtypeStruct((B, L, D), x.dtype),
    )(x, pe[:L])


# speedup vs baseline: 1.7355x; 1.0401x over previous
import jax
import jax.numpy as jnp
from jax.experimental import pallas as pl


L_BLK = 2048


def _body(x_ref, pe_ref, o_ref):
    o_ref[0] = x_ref[0] + pe_ref[...]


def kernel(x, pe):
    B, L, D = x.shape
    n_l = L // L_BLK
    return pl.pallas_call(
        _body,
        grid=(n_l, B),
        in_specs=[
            pl.BlockSpec((1, L_BLK, D), lambda i, b: (b, i, 0)),
            pl.BlockSpec((L_BLK, D), lambda i, b: (i, 0)),
        ],
        out_specs=pl.BlockSpec((1, L_BLK, D), lambda i, b: (b, i, 0)),
        out_shape=jax.ShapeDtypeStruct((B, L, D), x.dtype),
    )(x, pe[:L])
